# baseline (speedup 1.0000x reference)
import functools

import jax
import jax.numpy as jnp
from jax import lax
from jax.experimental import pallas as pl
from jax.experimental.pallas import tpu as pltpu
from jax.experimental.pallas import tpu_sc as plsc

SIZE = 262144
STATE_DIM = 128
ACTION_DIM = 32
BATCH = 16384

NUM_CORES = 2
NUM_SUBCORES = 16
NW = NUM_CORES * NUM_SUBCORES
B_PER_W = BATCH // NW
CHUNK = 128
NCHUNK = B_PER_W // CHUNK
DEPTH = 2

ABLK = (SIZE // 128) * 1024
ALAST = (SIZE // 128 - 1) * 1024 + 128


def kernel(states, actions, rewards, states_next, dones, indices):
    idx = indices.astype(jnp.int32)
    araw = actions.reshape(SIZE // 128, 128, 4, 8)
    araw = araw.transpose(2, 0, 3, 1).reshape(ACTION_DIM * SIZE)
    mesh = plsc.VectorSubcoreMesh(core_axis_name="c", subcore_axis_name="s")

    @functools.partial(
        pl.kernel,
        mesh=mesh,
        out_type=[
            jax.ShapeDtypeStruct((BATCH, STATE_DIM), jnp.float32),
            jax.ShapeDtypeStruct((ACTION_DIM, BATCH), jnp.float32),
            jax.ShapeDtypeStruct((BATCH,), jnp.float32),
            jax.ShapeDtypeStruct((BATCH, STATE_DIM), jnp.float32),
            jax.ShapeDtypeStruct((BATCH,), jnp.float32),
        ],
        scratch_types=[
            pltpu.VMEM((B_PER_W,), jnp.int32),
            pltpu.VMEM((B_PER_W,), jnp.int32),
            pltpu.VMEM((DEPTH, CHUNK, STATE_DIM), jnp.float32),
            pltpu.VMEM((DEPTH, CHUNK, STATE_DIM), jnp.float32),
            pltpu.VMEM((DEPTH, CHUNK), jnp.float32),
            pltpu.VMEM((DEPTH, CHUNK), jnp.float32),
            pltpu.VMEM((ACTION_DIM, B_PER_W), jnp.float32),
            [pltpu.SemaphoreType.DMA] * (4 * DEPTH + 1),
        ],
    )
    def gather_kernel(s_hbm, ar_hbm, r_hbm, sn_hbm, d_hbm, idx_hbm,
                      s_out, at_out, r_out, sn_out, d_out,
                      idx_v, h_v, s_v, sn_v, rw_v, dn_v, a_vt, sems):
        wid = lax.axis_index("s") * NUM_CORES + lax.axis_index("c")
        base = wid * B_PER_W
        pltpu.sync_copy(idx_hbm.at[pl.ds(base, B_PER_W)], idx_v)
        for t in range(B_PER_W // 16):
            v = idx_v[pl.ds(16 * t, 16)]
            h_v[pl.ds(16 * t, 16)] = (
                lax.shift_left(lax.shift_right_logical(v, 7), 10)
                + lax.bitwise_and(v, 127))

        def afire(j):
            ih = h_v.at[pl.ds(j * CHUNK, CHUNK)]
            return [
                pltpu.async_copy(
                    ar_hbm.at[pl.ds((c >> 3) * ABLK + (c & 7) * 128,
                                    ALAST)].at[ih],
                    a_vt.at[c, pl.ds(j * CHUNK, CHUNK)], sems[4 * DEPTH])
                for c in range(ACTION_DIM)
            ]

        def fire(j):
            b = j % DEPTH
            ic = idx_v.at[pl.ds(j * CHUNK, CHUNK)]
            cs = pltpu.async_copy(s_hbm.at[ic], s_v.at[b], sems[b])
            csn = pltpu.async_copy(sn_hbm.at[ic], sn_v.at[b],
                                   sems[DEPTH + b])
            cr = pltpu.async_copy(r_hbm.at[ic], rw_v.at[b],
                                  sems[2 * DEPTH + b])
            cd = pltpu.async_copy(d_hbm.at[ic], dn_v.at[b],
                                  sems[3 * DEPTH + b])
            return cs, csn, cr, cd

        ainflight = [afire(0), afire(1)]
        inflight = [fire(j) for j in range(DEPTH - 1)]
        for j in range(NCHUNK):
            if j + DEPTH - 1 < NCHUNK:
                inflight.append(fire(j + DEPTH - 1))
            if j + 2 < NCHUNK:
                ainflight.append(afire(j + 2))
            cs, csn, cr, cd = inflight.pop(0)
            b = j % DEPTH
            off = base + j * CHUNK
            cs.wait()
            pltpu.sync_copy(s_v.at[b], s_out.at[pl.ds(off, CHUNK)])
            csn.wait()
            pltpu.sync_copy(sn_v.at[b], sn_out.at[pl.ds(off, CHUNK)])
            cr.wait()
            pltpu.sync_copy(rw_v.at[b], r_out.at[pl.ds(off, CHUNK)])
            cd.wait()
            pltpu.sync_copy(dn_v.at[b], d_out.at[pl.ds(off, CHUNK)])
            for c in ainflight.pop(0):
                c.wait()
        pltpu.sync_copy(a_vt, at_out.at[:, pl.ds(base, B_PER_W)])

    s, a_t, r, sn, d = gather_kernel(states, araw, rewards,
                                     states_next, dones, idx)
    return (s, a_t.T, r, sn, d)


# speedup vs baseline: 1.0192x; 1.0192x over previous
import functools

import jax
import jax.numpy as jnp
from jax import lax
from jax.experimental import pallas as pl
from jax.experimental.pallas import tpu as pltpu
from jax.experimental.pallas import tpu_sc as plsc

SIZE = 262144
STATE_DIM = 128
ACTION_DIM = 32
BATCH = 16384

NUM_CORES = 2
NUM_SUBCORES = 16
NW = NUM_CORES * NUM_SUBCORES
B_PER_W = BATCH // NW
CHUNK = 128
NCHUNK = B_PER_W // CHUNK
DEPTH = 2

ABLK = (SIZE // 128) * 1024
ALAST = (SIZE // 128 - 1) * 1024 + 128


def kernel(states, actions, rewards, states_next, dones, indices):
    idx = indices.astype(jnp.int32)
    araw = actions.reshape(SIZE // 128, 128, 4, 8)
    araw = araw.transpose(2, 0, 3, 1).reshape(ACTION_DIM * SIZE)
    mesh = plsc.VectorSubcoreMesh(core_axis_name="c", subcore_axis_name="s")

    @functools.partial(
        pl.kernel,
        mesh=mesh,
        out_type=[
            jax.ShapeDtypeStruct((BATCH, STATE_DIM), jnp.float32),
            jax.ShapeDtypeStruct((ACTION_DIM, BATCH), jnp.float32),
            jax.ShapeDtypeStruct((BATCH,), jnp.float32),
            jax.ShapeDtypeStruct((BATCH, STATE_DIM), jnp.float32),
            jax.ShapeDtypeStruct((BATCH,), jnp.float32),
        ],
        scratch_types=[
            pltpu.VMEM((B_PER_W,), jnp.int32),
            pltpu.VMEM((B_PER_W,), jnp.int32),
            pltpu.VMEM((DEPTH, CHUNK, STATE_DIM), jnp.float32),
            pltpu.VMEM((DEPTH, CHUNK, STATE_DIM), jnp.float32),
            pltpu.VMEM((DEPTH, CHUNK), jnp.float32),
            pltpu.VMEM((DEPTH, CHUNK), jnp.float32),
            pltpu.VMEM((DEPTH, ACTION_DIM, CHUNK), jnp.float32),
            [pltpu.SemaphoreType.DMA] * (5 * DEPTH),
        ],
    )
    def gather_kernel(s_hbm, ar_hbm, r_hbm, sn_hbm, d_hbm, idx_hbm,
                      s_out, at_out, r_out, sn_out, d_out,
                      idx_v, h_v, s_v, sn_v, rw_v, dn_v, a_vt, sems):
        wid = lax.axis_index("s") * NUM_CORES + lax.axis_index("c")
        base = wid * B_PER_W
        pltpu.sync_copy(idx_hbm.at[pl.ds(base, B_PER_W)], idx_v)
        for t in range(B_PER_W // 16):
            v = idx_v[pl.ds(16 * t, 16)]
            h_v[pl.ds(16 * t, 16)] = (
                lax.shift_left(lax.shift_right_logical(v, 7), 10)
                + lax.bitwise_and(v, 127))

        def fire(j):
            b = j % DEPTH
            ic = idx_v.at[pl.ds(j * CHUNK, CHUNK)]
            ih = h_v.at[pl.ds(j * CHUNK, CHUNK)]
            cs = pltpu.async_copy(s_hbm.at[ic], s_v.at[b], sems[b])
            csn = pltpu.async_copy(sn_hbm.at[ic], sn_v.at[b],
                                   sems[DEPTH + b])
            cr = pltpu.async_copy(r_hbm.at[ic], rw_v.at[b],
                                  sems[2 * DEPTH + b])
            cd = pltpu.async_copy(d_hbm.at[ic], dn_v.at[b],
                                  sems[3 * DEPTH + b])
            acs = [
                pltpu.async_copy(
                    ar_hbm.at[pl.ds((c >> 3) * ABLK + (c & 7) * 128,
                                    ALAST)].at[ih],
                    a_vt.at[b, c], sems[4 * DEPTH + b])
                for c in range(ACTION_DIM)
            ]
            return cs, csn, cr, cd, acs

        inflight = [fire(j) for j in range(DEPTH - 1)]
        for j in range(NCHUNK):
            if j + DEPTH - 1 < NCHUNK:
                inflight.append(fire(j + DEPTH - 1))
            cs, csn, cr, cd, acs = inflight.pop(0)
            b = j % DEPTH
            off = base + j * CHUNK
            cs.wait()
            pltpu.sync_copy(s_v.at[b], s_out.at[pl.ds(off, CHUNK)])
            csn.wait()
            pltpu.sync_copy(sn_v.at[b], sn_out.at[pl.ds(off, CHUNK)])
            cr.wait()
            pltpu.sync_copy(rw_v.at[b], r_out.at[pl.ds(off, CHUNK)])
            cd.wait()
            pltpu.sync_copy(dn_v.at[b], d_out.at[pl.ds(off, CHUNK)])
            for c in range(ACTION_DIM):
                acs[c].wait()
            pltpu.sync_copy(a_vt.at[b], at_out.at[:, pl.ds(off, CHUNK)])

    s, a_t, r, sn, d = gather_kernel(states, araw, rewards,
                                     states_next, dones, idx)
    return (s, a_t.T, r, sn, d)
